# baseline (speedup 1.0000x reference)
import functools

import jax
import jax.numpy as jnp
from jax import lax
from jax.experimental import pallas as pl
from jax.experimental.pallas import tpu as pltpu
from jax.experimental.pallas import tpu_sc as plsc

_NPOINT = 8192
_NSAMPLE = 32
_B = 2
_NTILES = 16


def _fps_tc_body(npoint, n8, x_ref, y_ref, z_ref, out_ref, dl):
    x = x_ref[0]
    y = y_ref[0]
    z = z_ref[0]
    dl[...] = jnp.full((8, n8), 1e10, jnp.float32)
    rows = jax.lax.broadcasted_iota(jnp.int32, (8, n8), 0)
    cols = jax.lax.broadcasted_iota(jnp.int32, (8, n8), 1)
    flat = rows * n8 + cols
    lane = jax.lax.iota(jnp.int32, 128)
    n = 8 * n8

    def body(i, carry):
        cx, cy, cz, gi, pend = carry
        pend = jnp.where(lane == (i & 127), gi, pend)

        @pl.when((i & 127) == 127)
        def _():
            out_ref[0, 0, pl.ds(pl.multiple_of(i - 127, 128), 128)] = pend

        dx = x - cx
        dy = y - cy
        dz = z - cz
        d = (dx * dx + dy * dy) + dz * dz
        dn = jnp.minimum(dl[...], d)
        dl[...] = dn
        mmax = jnp.max(dn)
        eqm = dn == mmax
        g2 = jnp.min(jnp.where(eqm, flat, n))
        one = eqm & (flat == g2)
        zero = jnp.zeros((8, n8), jnp.float32)
        nx = jnp.sum(jnp.where(one, x, zero))
        ny = jnp.sum(jnp.where(one, y, zero))
        nz = jnp.sum(jnp.where(one, z, zero))
        return (nx, ny, nz, g2, pend)

    x0 = x[0, 0]
    y0 = y[0, 0]
    z0 = z[0, 0]
    lax.fori_loop(0, npoint, body,
                  (x0, y0, z0, jnp.int32(0), jnp.zeros((128,), jnp.int32)))


def _fps_tc(pc, npoint):
    n = pc.shape[2]
    n8 = n // 8
    coords = pc.reshape(_B, 3, 8, n8)
    xs = coords[:, 0]
    ys = coords[:, 1]
    zs = coords[:, 2]
    body = functools.partial(_fps_tc_body, npoint, n8)
    out = pl.pallas_call(
        body,
        grid=(_B,),
        in_specs=[
            pl.BlockSpec((1, 8, n8), lambda b: (b, 0, 0)),
            pl.BlockSpec((1, 8, n8), lambda b: (b, 0, 0)),
            pl.BlockSpec((1, 8, n8), lambda b: (b, 0, 0)),
        ],
        out_specs=pl.BlockSpec((1, 1, npoint), lambda b: (b, 0, 0)),
        out_shape=jax.ShapeDtypeStruct((_B, 1, npoint), jnp.int32),
        scratch_shapes=[pltpu.VMEM((8, n8), jnp.float32)],
    )(xs, ys, zs)
    return out.reshape(_B, npoint)


def _fps_sc_body(npoint, n_per_tile,
                 x_hbm, y_hbm, z_hbm, out_hbm,
                 xl, yl, zl, dl, tmp, buf, loc, idx_buf, tab):
    b = lax.axis_index("c")
    s = lax.axis_index("s")
    base = s * n_per_tile
    nchunks = n_per_tile // 16

    pltpu.sync_copy(x_hbm.at[b, 0, pl.ds(base, n_per_tile)], xl)
    pltpu.sync_copy(y_hbm.at[b, 0, pl.ds(base, n_per_tile)], yl)
    pltpu.sync_copy(z_hbm.at[b, 0, pl.ds(base, n_per_tile)], zl)

    big = jnp.full((16,), 1e10, dtype=jnp.float32)
    for j in range(nchunks):
        dl[pl.ds(j * 16, 16)] = big

    pltpu.sync_copy(x_hbm.at[b, 0, pl.ds(0, 16)], tmp)
    cx0 = tmp[...][0]
    pltpu.sync_copy(y_hbm.at[b, 0, pl.ds(0, 16)], tmp)
    cy0 = tmp[...][0]
    pltpu.sync_copy(z_hbm.at[b, 0, pl.ds(0, 16)], tmp)
    cz0 = tmp[...][0]

    lane = lax.iota(jnp.int32, 16)
    zero16f = jnp.zeros((16,), jnp.float32)

    def body(i, carry):
        cx, cy, cz, gi, pend = carry
        pend = jnp.where(lane == (i & 15), gi, pend)

        @pl.when((s == 0) & ((i & 15) == 15))
        def _():
            idx_buf[pl.ds(pl.multiple_of(i - 15, 16), 16)] = pend

        cxv = jnp.full((16,), cx, dtype=jnp.float32)
        cyv = jnp.full((16,), cy, dtype=jnp.float32)
        czv = jnp.full((16,), cz, dtype=jnp.float32)

        m = jnp.full((16,), -1.0, dtype=jnp.float32)
        a = jnp.zeros((16,), dtype=jnp.int32)
        mx = zero16f
        my = zero16f
        mz = zero16f
        for j in range(nchunks):
            sl = pl.ds(j * 16, 16)
            xv = xl[sl]
            yv = yl[sl]
            zv = zl[sl]
            dx = xv - cxv
            dy = yv - cyv
            dz = zv - czv
            d = (dx * dx + dy * dy) + dz * dz
            dn = jnp.minimum(dl[sl], d)
            dl[sl] = dn
            upd = dn > m
            m = jnp.where(upd, dn, m)
            a = jnp.where(upd, lane + (j * 16), a)
            mx = jnp.where(upd, xv, mx)
            my = jnp.where(upd, yv, my)
            mz = jnp.where(upd, zv, mz)

        mmax = jnp.max(m)
        win = (m == mmax)
        amin = jnp.min(jnp.where(win, a, jnp.int32(2 * n_per_tile)))
        one = win & (a == amin)
        xa = jnp.sum(jnp.where(one, mx, zero16f))
        ya = jnp.sum(jnp.where(one, my, zero16f))
        za = jnp.sum(jnp.where(one, mz, zero16f))

        row = jnp.full((16,), mmax, jnp.float32)
        row = jnp.where(lane == 1, (amin + base).astype(jnp.float32), row)
        row = jnp.where(lane == 2, xa, row)
        row = jnp.where(lane == 3, ya, row)
        row = jnp.where(lane == 4, za, row)
        buf[...] = row
        pltpu.sync_copy(buf, tab.at[s])
        plsc.subcore_barrier()
        pltpu.sync_copy(tab, loc)
        plsc.subcore_barrier()

        r0 = loc[0]
        bm, ba, bx, by, bz = r0[0], r0[1], r0[2], r0[3], r0[4]
        for r in range(1, _NTILES):
            rr = loc[r]
            take = rr[0] > bm
            bm = jnp.where(take, rr[0], bm)
            ba = jnp.where(take, rr[1], ba)
            bx = jnp.where(take, rr[2], bx)
            by = jnp.where(take, rr[3], by)
            bz = jnp.where(take, rr[4], bz)
        return (bx, by, bz, ba.astype(jnp.int32), pend)

    lax.fori_loop(0, npoint, body,
                  (cx0, cy0, cz0, jnp.int32(0),
                   jnp.zeros((16,), jnp.int32)))

    @pl.when(s == 0)
    def _():
        pltpu.sync_copy(idx_buf, out_hbm.at[b])


def _fps_sc(pc, npoint):
    n = pc.shape[2]
    n_per_tile = n // _NTILES
    body = functools.partial(_fps_sc_body, npoint, n_per_tile)
    fn = pl.kernel(
        body,
        out_type=jax.ShapeDtypeStruct((_B, npoint), jnp.int32),
        mesh=plsc.VectorSubcoreMesh(core_axis_name="c", subcore_axis_name="s"),
        compiler_params=pltpu.CompilerParams(needs_layout_passes=False),
        scratch_types=[
            pltpu.VMEM((n_per_tile,), jnp.float32),
            pltpu.VMEM((n_per_tile,), jnp.float32),
            pltpu.VMEM((n_per_tile,), jnp.float32),
            pltpu.VMEM((n_per_tile,), jnp.float32),
            pltpu.VMEM((16,), jnp.float32),
            pltpu.VMEM((16,), jnp.float32),
            pltpu.VMEM((_NTILES, 16), jnp.float32),
            pltpu.VMEM((npoint,), jnp.int32),
            pltpu.VMEM_SHARED((_NTILES, 16), jnp.float32),
        ],
    )
    return fn(pc[:, 0:1], pc[:, 1:2], pc[:, 2:3])


def _square_distance(src, dst):
    d = -2.0 * jnp.matmul(src, jnp.transpose(dst, (0, 2, 1)))
    d = d + jnp.sum(src ** 2, -1)[:, :, None]
    d = d + jnp.sum(dst ** 2, -1)[:, None, :]
    return d


def _index_points(points, idx):
    return jax.vmap(lambda p, i: p[i])(points, idx)


def _fps(xyz, npoint):
    Bb, N, _ = xyz.shape
    def body(i, state):
        centroids, distance, farthest = state
        centroids = centroids.at[:, i].set(farthest)
        centroid = jnp.take_along_axis(xyz, farthest[:, None, None].astype(jnp.int32), axis=1)
        dist = jnp.sum((xyz - centroid) ** 2, -1)
        distance = jnp.minimum(distance, dist)
        farthest = jnp.argmax(distance, axis=-1).astype(jnp.int32)
        return (centroids, distance, farthest)
    centroids = jnp.zeros((Bb, npoint), dtype=jnp.int32)
    distance = jnp.full((Bb, N), 1e10, dtype=xyz.dtype)
    farthest = jnp.zeros((Bb,), dtype=jnp.int32)
    centroids, _, _ = jax.lax.fori_loop(0, npoint, body, (centroids, distance, farthest))
    return centroids


def _knn(nsample, xyz, new_xyz):
    dist = _square_distance(new_xyz, xyz)
    _, idx = jax.lax.top_k(-dist, nsample)
    return idx


def _sa(xyz, points, npoint, nsample, Ws, bs):
    xyz_t = jnp.transpose(xyz, (0, 2, 1))
    pts_t = jnp.transpose(points, (0, 2, 1))
    fps_idx = _fps_tc(xyz, npoint)
    new_xyz = _index_points(xyz_t, fps_idx)
    idx = _knn(nsample, xyz_t, new_xyz)
    grouped_xyz = _index_points(xyz_t, idx)
    grouped_xyz_norm = grouped_xyz - new_xyz[:, :, None, :]
    grouped_pts = _index_points(pts_t, idx)
    new_points = jnp.concatenate([grouped_xyz_norm, grouped_pts], axis=-1)
    x = jnp.transpose(new_points, (0, 3, 2, 1))
    for W, b in zip(Ws, bs):
        x = jnp.einsum('oc,bcks->boks', W, x) + b[None, :, None, None]
        mean = jnp.mean(x, axis=(2, 3), keepdims=True)
        var = jnp.var(x, axis=(2, 3), keepdims=True)
        x = (x - mean) / jnp.sqrt(var + 1e-5)
        x = jax.nn.relu(x)
    new_feat = jnp.max(x, axis=2)
    return jnp.transpose(new_xyz, (0, 2, 1)), new_feat, fps_idx


def _identity_body(x_ref, o_ref):
    o_ref[...] = x_ref[...]


def _pallas_identity(x):
    return pl.pallas_call(
        _identity_body,
        out_shape=jax.ShapeDtypeStruct(x.shape, x.dtype),
    )(x)


def kernel(pc, feature, W1_0, b1_0, W1_1, b1_1, W1_2, b1_2, W2_0, b2_0, W2_1, b2_1, W2_2, b2_2):
    pc_l1, feat_l1, fps_idx1 = _sa(pc, feature, _NPOINT // 2, _NSAMPLE,
                                   [W1_0, W1_1, W1_2], [b1_0, b1_1, b1_2])
    pc_l2, feat_l2, fps_idx2 = _sa(pc_l1, feat_l1, _NPOINT // 4, _NSAMPLE,
                                   [W2_0, W2_1, W2_2], [b2_0, b2_1, b2_2])
    feat_l2 = _pallas_identity(feat_l2)
    return (pc, pc_l1, pc_l2, feat_l2, fps_idx1, fps_idx2)


# speedup vs baseline: 1.6985x; 1.6985x over previous
import functools

import jax
import jax.numpy as jnp
from jax import lax
from jax.experimental import pallas as pl
from jax.experimental.pallas import tpu as pltpu
from jax.experimental.pallas import tpu_sc as plsc

_NPOINT = 8192
_NSAMPLE = 32
_B = 2
_NTILES = 16


def _fps_tc_body(npoint, n8, x_ref, y_ref, z_ref, out_ref, dl):
    x = x_ref[0]
    y = y_ref[0]
    z = z_ref[0]
    dl[...] = jnp.full((8, n8), 1e10, jnp.float32)
    rows = jax.lax.broadcasted_iota(jnp.int32, (8, n8), 0)
    cols = jax.lax.broadcasted_iota(jnp.int32, (8, n8), 1)
    flat = rows * n8 + cols
    lane = jax.lax.iota(jnp.int32, 128)
    n = 8 * n8

    def body(i, carry):
        cx, cy, cz, gi, pend = carry
        pend = jnp.where(lane == (i & 127), gi, pend)

        @pl.when((i & 127) == 127)
        def _():
            out_ref[0, 0, pl.ds(pl.multiple_of(i - 127, 128), 128)] = pend

        dx = x - cx
        dy = y - cy
        dz = z - cz
        d = (dx * dx + dy * dy) + dz * dz
        dn = jnp.minimum(dl[...], d)
        dl[...] = dn
        mmax = jnp.max(dn)
        eqm = dn == mmax
        g2 = jnp.min(jnp.where(eqm, flat, n))
        one = eqm & (flat == g2)
        zero = jnp.zeros((8, n8), jnp.float32)
        nx = jnp.sum(jnp.where(one, x, zero))
        ny = jnp.sum(jnp.where(one, y, zero))
        nz = jnp.sum(jnp.where(one, z, zero))
        return (nx, ny, nz, g2, pend)

    x0 = x[0, 0]
    y0 = y[0, 0]
    z0 = z[0, 0]
    lax.fori_loop(0, npoint, body,
                  (x0, y0, z0, jnp.int32(0), jnp.zeros((128,), jnp.int32)))


def _fps_tc(pc, npoint):
    n = pc.shape[2]
    n8 = n // 8
    coords = pc.reshape(_B, 3, 8, n8)
    xs = coords[:, 0]
    ys = coords[:, 1]
    zs = coords[:, 2]
    body = functools.partial(_fps_tc_body, npoint, n8)
    out = pl.pallas_call(
        body,
        grid=(_B,),
        in_specs=[
            pl.BlockSpec((1, 8, n8), lambda b: (b, 0, 0)),
            pl.BlockSpec((1, 8, n8), lambda b: (b, 0, 0)),
            pl.BlockSpec((1, 8, n8), lambda b: (b, 0, 0)),
        ],
        out_specs=pl.BlockSpec((1, 1, npoint), lambda b: (b, 0, 0)),
        out_shape=jax.ShapeDtypeStruct((_B, 1, npoint), jnp.int32),
        scratch_shapes=[pltpu.VMEM((8, n8), jnp.float32)],
    )(xs, ys, zs)
    return out.reshape(_B, npoint)


def _fps_sc_body(npoint, n_per_tile,
                 x_hbm, y_hbm, z_hbm, out_hbm,
                 xl, yl, zl, dl, tmp, buf, loc, idx_buf, tab):
    b = lax.axis_index("c")
    s = lax.axis_index("s")
    base = s * n_per_tile
    nchunks = n_per_tile // 16

    pltpu.sync_copy(x_hbm.at[b, 0, pl.ds(base, n_per_tile)], xl)
    pltpu.sync_copy(y_hbm.at[b, 0, pl.ds(base, n_per_tile)], yl)
    pltpu.sync_copy(z_hbm.at[b, 0, pl.ds(base, n_per_tile)], zl)

    big = jnp.full((16,), 1e10, dtype=jnp.float32)
    for j in range(nchunks):
        dl[pl.ds(j * 16, 16)] = big

    pltpu.sync_copy(x_hbm.at[b, 0, pl.ds(0, 16)], tmp)
    cx0 = tmp[...][0]
    pltpu.sync_copy(y_hbm.at[b, 0, pl.ds(0, 16)], tmp)
    cy0 = tmp[...][0]
    pltpu.sync_copy(z_hbm.at[b, 0, pl.ds(0, 16)], tmp)
    cz0 = tmp[...][0]

    lane = lax.iota(jnp.int32, 16)
    zero16f = jnp.zeros((16,), jnp.float32)

    def body(i, carry):
        cx, cy, cz, gi, pend = carry
        pend = jnp.where(lane == (i & 15), gi, pend)

        @pl.when((s == 0) & ((i & 15) == 15))
        def _():
            idx_buf[pl.ds(pl.multiple_of(i - 15, 16), 16)] = pend

        cxv = jnp.full((16,), cx, dtype=jnp.float32)
        cyv = jnp.full((16,), cy, dtype=jnp.float32)
        czv = jnp.full((16,), cz, dtype=jnp.float32)

        m = jnp.full((16,), -1.0, dtype=jnp.float32)
        a = jnp.zeros((16,), dtype=jnp.int32)
        mx = zero16f
        my = zero16f
        mz = zero16f
        for j in range(nchunks):
            sl = pl.ds(j * 16, 16)
            xv = xl[sl]
            yv = yl[sl]
            zv = zl[sl]
            dx = xv - cxv
            dy = yv - cyv
            dz = zv - czv
            d = (dx * dx + dy * dy) + dz * dz
            dn = jnp.minimum(dl[sl], d)
            dl[sl] = dn
            upd = dn > m
            m = jnp.where(upd, dn, m)
            a = jnp.where(upd, lane + (j * 16), a)
            mx = jnp.where(upd, xv, mx)
            my = jnp.where(upd, yv, my)
            mz = jnp.where(upd, zv, mz)

        mmax = jnp.max(m)
        win = (m == mmax)
        amin = jnp.min(jnp.where(win, a, jnp.int32(2 * n_per_tile)))
        one = win & (a == amin)
        xa = jnp.sum(jnp.where(one, mx, zero16f))
        ya = jnp.sum(jnp.where(one, my, zero16f))
        za = jnp.sum(jnp.where(one, mz, zero16f))

        row = jnp.full((16,), mmax, jnp.float32)
        row = jnp.where(lane == 1, (amin + base).astype(jnp.float32), row)
        row = jnp.where(lane == 2, xa, row)
        row = jnp.where(lane == 3, ya, row)
        row = jnp.where(lane == 4, za, row)
        buf[...] = row
        pltpu.sync_copy(buf, tab.at[s])
        plsc.subcore_barrier()
        pltpu.sync_copy(tab, loc)
        plsc.subcore_barrier()

        r0 = loc[0]
        bm, ba, bx, by, bz = r0[0], r0[1], r0[2], r0[3], r0[4]
        for r in range(1, _NTILES):
            rr = loc[r]
            take = rr[0] > bm
            bm = jnp.where(take, rr[0], bm)
            ba = jnp.where(take, rr[1], ba)
            bx = jnp.where(take, rr[2], bx)
            by = jnp.where(take, rr[3], by)
            bz = jnp.where(take, rr[4], bz)
        return (bx, by, bz, ba.astype(jnp.int32), pend)

    lax.fori_loop(0, npoint, body,
                  (cx0, cy0, cz0, jnp.int32(0),
                   jnp.zeros((16,), jnp.int32)))

    @pl.when(s == 0)
    def _():
        pltpu.sync_copy(idx_buf, out_hbm.at[b])


def _fps_sc(pc, npoint):
    n = pc.shape[2]
    n_per_tile = n // _NTILES
    body = functools.partial(_fps_sc_body, npoint, n_per_tile)
    fn = pl.kernel(
        body,
        out_type=jax.ShapeDtypeStruct((_B, npoint), jnp.int32),
        mesh=plsc.VectorSubcoreMesh(core_axis_name="c", subcore_axis_name="s"),
        compiler_params=pltpu.CompilerParams(needs_layout_passes=False),
        scratch_types=[
            pltpu.VMEM((n_per_tile,), jnp.float32),
            pltpu.VMEM((n_per_tile,), jnp.float32),
            pltpu.VMEM((n_per_tile,), jnp.float32),
            pltpu.VMEM((n_per_tile,), jnp.float32),
            pltpu.VMEM((16,), jnp.float32),
            pltpu.VMEM((16,), jnp.float32),
            pltpu.VMEM((_NTILES, 16), jnp.float32),
            pltpu.VMEM((npoint,), jnp.int32),
            pltpu.VMEM_SHARED((_NTILES, 16), jnp.float32),
        ],
    )
    return fn(pc[:, 0:1], pc[:, 1:2], pc[:, 2:3])


_INF = jnp.float32(jnp.inf)


def _bf16_round(x):
    bits = plsc.bitcast(x, jnp.uint32)
    rounded = (bits + jnp.uint32(0x7FFF) + ((bits >> 16) & jnp.uint32(1))) \
        & jnp.uint32(0xFFFF0000)
    return plsc.bitcast(rounded, jnp.float32)


def _bf16_round_scalar(x):
    bits = lax.bitcast_convert_type(x, jnp.uint32)
    rounded = (bits + jnp.uint32(0x7FFF) + ((bits >> 16) & jnp.uint32(1))) \
        & jnp.uint32(0xFFFF0000)
    return lax.bitcast_convert_type(rounded, jnp.float32)


def _knn_sc_body(n, s_per_tile, nsample,
                 x_hbm, y_hbm, z_hbm, fidx_hbm, out_hbm,
                 xl, yl, zl, pn, qx, qy, qz, qi, cbuf_d, cbuf_i,
                 sel_d, sel_i, sem):
    b = lax.axis_index("c")
    s = lax.axis_index("s")
    lane = lax.iota(jnp.int32, 16)
    qbase = s * s_per_tile
    nblocks = n // 128
    cap_chunks = 16

    xf = x_hbm.at[b, 0]
    yf = y_hbm.at[b, 0]
    zf = z_hbm.at[b, 0]
    pltpu.sync_copy(xf, xl)
    pltpu.sync_copy(yf, yl)
    pltpu.sync_copy(zf, zl)
    pltpu.sync_copy(fidx_hbm.at[b, 0, pl.ds(qbase, s_per_tile)], qi)
    pltpu.async_copy(xf.at[qi], qx, sem).wait()
    pltpu.async_copy(yf.at[qi], qy, sem).wait()
    pltpu.async_copy(zf.at[qi], qz, sem).wait()

    def pn_body(j, _):
        sl = pl.ds(pl.multiple_of(j * 16, 16), 16)
        xv = xl[sl]
        yv = yl[sl]
        zv = zl[sl]
        pn[sl] = (xv * xv + yv * yv) + zv * zv
        xl[sl] = _bf16_round(xv)
        yl[sl] = _bf16_round(yv)
        zl[sl] = _bf16_round(zv)
        return 0
    lax.fori_loop(0, n // 16, pn_body, 0)

    zero16f = jnp.zeros((16,), jnp.float32)
    inf16 = jnp.full((16,), _INF, jnp.float32)
    big_i = jnp.full((16,), jnp.int32(2 ** 30), jnp.int32)

    def rebuild():
        def ext(k, carry):
            pend_d, pend_i, _ = carry
            m = inf16
            for c in range(cap_chunks):
                m = jnp.minimum(m, cbuf_d[pl.ds(c * 16, 16)])
            mdb = jnp.min(m)
            pv = big_i
            for c in range(cap_chunks):
                pv = jnp.minimum(
                    pv, jnp.where(cbuf_d[pl.ds(c * 16, 16)] == mdb,
                                  lane + (c * 16), big_i))
            posb = jnp.min(pv)
            cb = pl.multiple_of(posb - (posb & 15), 16)
            lh = lane == (posb & 15)
            dchunk = cbuf_d[pl.ds(cb, 16)]
            ichunk = cbuf_i[pl.ds(cb, 16)]
            iv = jnp.sum(jnp.where(lh, ichunk, jnp.zeros((16,), jnp.int32)))
            cbuf_d[pl.ds(cb, 16)] = jnp.where(lh, inf16, dchunk)
            pend_d = jnp.where(lane == (k & 15), mdb, pend_d)
            pend_i = jnp.where(lane == (k & 15), iv, pend_i)

            @pl.when((k & 15) == 15)
            def _():
                fl = pl.ds(pl.multiple_of(k - 15, 16), 16)
                sel_d[fl] = pend_d
                sel_i[fl] = pend_i
            return (pend_d, pend_i, mdb)

        _, _, thr_new = lax.fori_loop(
            0, nsample, ext,
            (zero16f, jnp.zeros((16,), jnp.int32), _INF))
        for c in range(nsample // 16):
            sl = pl.ds(c * 16, 16)
            cbuf_d[sl] = sel_d[sl]
            cbuf_i[sl] = sel_i[sl]
        return thr_new

    def query_body(q, _):
        bq = pl.multiple_of(q - (q & 15), 16)
        lsel = lane == (q & 15)
        qxs = jnp.sum(jnp.where(lsel, qx[pl.ds(bq, 16)], zero16f))
        qys = jnp.sum(jnp.where(lsel, qy[pl.ds(bq, 16)], zero16f))
        qzs = jnp.sum(jnp.where(lsel, qz[pl.ds(bq, 16)], zero16f))
        qn = (qxs * qxs + qys * qys) + qzs * qzs
        qxs = _bf16_round_scalar(qxs)
        qys = _bf16_round_scalar(qys)
        qzs = _bf16_round_scalar(qzs)

        def block_body(j, carry):
            thr, cnt = carry
            for c in range(8):
                off = j * 128 + c * 16
                sl = pl.ds(pl.multiple_of(off, 16), 16)
                xv = xl[sl]
                yv = yl[sl]
                zv = zl[sl]
                pnv = pn[sl]
                cross = (xv * qxs + yv * qys) + zv * qzs
                d = (cross * jnp.float32(-2.0) + qn) + pnv
                idxv = lane + off
                msk = d < thr
                plsc.store_compressed(cbuf_d.at[pl.ds(cnt, 16)], d, mask=msk)
                plsc.store_compressed(cbuf_i.at[pl.ds(cnt, 16)], idxv,
                                      mask=msk)
                cnt = cnt + plsc.all_reduce_population_count(msk)[0]
            return lax.cond(cnt > 128,
                            lambda: (rebuild(), jnp.int32(nsample)),
                            lambda: (thr, cnt))

        lax.fori_loop(0, nblocks, block_body, (_INF, jnp.int32(0)))
        rebuild()
        pltpu.sync_copy(
            sel_i.at[pl.ds(0, nsample)],
            out_hbm.at[b, 0, pl.ds((qbase + q) * nsample, nsample)])
        return 0

    lax.fori_loop(0, s_per_tile, query_body, 0)


def _knn_sc(pc, fps_idx, nsample):
    n = pc.shape[2]
    s_total = fps_idx.shape[1]
    s_per_tile = s_total // _NTILES
    body = functools.partial(_knn_sc_body, n, s_per_tile, nsample)
    fn = pl.kernel(
        body,
        out_type=jax.ShapeDtypeStruct((_B, 1, s_total * nsample), jnp.int32),
        mesh=plsc.VectorSubcoreMesh(core_axis_name="c", subcore_axis_name="s"),
        compiler_params=pltpu.CompilerParams(needs_layout_passes=False),
        scratch_types=[
            pltpu.VMEM((n,), jnp.float32),
            pltpu.VMEM((n,), jnp.float32),
            pltpu.VMEM((n,), jnp.float32),
            pltpu.VMEM((n,), jnp.float32),
            pltpu.VMEM((s_per_tile,), jnp.float32),
            pltpu.VMEM((s_per_tile,), jnp.float32),
            pltpu.VMEM((s_per_tile,), jnp.float32),
            pltpu.VMEM((s_per_tile,), jnp.int32),
            pltpu.VMEM((256,), jnp.float32),
            pltpu.VMEM((256,), jnp.int32),
            pltpu.VMEM((nsample,), jnp.float32),
            pltpu.VMEM((nsample,), jnp.int32),
            pltpu.SemaphoreType.DMA,
        ],
    )
    out = fn(pc[:, 0:1], pc[:, 1:2], pc[:, 2:3], fps_idx[:, None, :])
    return out.reshape(_B, s_total, nsample)


def _square_distance(src, dst):
    d = -2.0 * jnp.matmul(src, jnp.transpose(dst, (0, 2, 1)))
    d = d + jnp.sum(src ** 2, -1)[:, :, None]
    d = d + jnp.sum(dst ** 2, -1)[:, None, :]
    return d


def _index_points(points, idx):
    return jax.vmap(lambda p, i: p[i])(points, idx)


def _fps(xyz, npoint):
    Bb, N, _ = xyz.shape
    def body(i, state):
        centroids, distance, farthest = state
        centroids = centroids.at[:, i].set(farthest)
        centroid = jnp.take_along_axis(xyz, farthest[:, None, None].astype(jnp.int32), axis=1)
        dist = jnp.sum((xyz - centroid) ** 2, -1)
        distance = jnp.minimum(distance, dist)
        farthest = jnp.argmax(distance, axis=-1).astype(jnp.int32)
        return (centroids, distance, farthest)
    centroids = jnp.zeros((Bb, npoint), dtype=jnp.int32)
    distance = jnp.full((Bb, N), 1e10, dtype=xyz.dtype)
    farthest = jnp.zeros((Bb,), dtype=jnp.int32)
    centroids, _, _ = jax.lax.fori_loop(0, npoint, body, (centroids, distance, farthest))
    return centroids


def _knn(nsample, xyz, new_xyz):
    dist = _square_distance(new_xyz, xyz)
    _, idx = jax.lax.top_k(-dist, nsample)
    return idx


def _sa(xyz, points, npoint, nsample, Ws, bs):
    xyz_t = jnp.transpose(xyz, (0, 2, 1))
    pts_t = jnp.transpose(points, (0, 2, 1))
    fps_idx = _fps_tc(xyz, npoint)
    new_xyz = _index_points(xyz_t, fps_idx)
    idx = _knn_sc(xyz, fps_idx, nsample)
    grouped_xyz = _index_points(xyz_t, idx)
    grouped_xyz_norm = grouped_xyz - new_xyz[:, :, None, :]
    grouped_pts = _index_points(pts_t, idx)
    new_points = jnp.concatenate([grouped_xyz_norm, grouped_pts], axis=-1)
    x = jnp.transpose(new_points, (0, 3, 2, 1))
    for W, b in zip(Ws, bs):
        x = jnp.einsum('oc,bcks->boks', W, x) + b[None, :, None, None]
        mean = jnp.mean(x, axis=(2, 3), keepdims=True)
        var = jnp.var(x, axis=(2, 3), keepdims=True)
        x = (x - mean) / jnp.sqrt(var + 1e-5)
        x = jax.nn.relu(x)
    new_feat = jnp.max(x, axis=2)
    return jnp.transpose(new_xyz, (0, 2, 1)), new_feat, fps_idx


def _identity_body(x_ref, o_ref):
    o_ref[...] = x_ref[...]


def _pallas_identity(x):
    return pl.pallas_call(
        _identity_body,
        out_shape=jax.ShapeDtypeStruct(x.shape, x.dtype),
    )(x)


def kernel(pc, feature, W1_0, b1_0, W1_1, b1_1, W1_2, b1_2, W2_0, b2_0, W2_1, b2_1, W2_2, b2_2):
    pc_l1, feat_l1, fps_idx1 = _sa(pc, feature, _NPOINT // 2, _NSAMPLE,
                                   [W1_0, W1_1, W1_2], [b1_0, b1_1, b1_2])
    pc_l2, feat_l2, fps_idx2 = _sa(pc_l1, feat_l1, _NPOINT // 4, _NSAMPLE,
                                   [W2_0, W2_1, W2_2], [b2_0, b2_1, b2_2])
    feat_l2 = _pallas_identity(feat_l2)
    return (pc, pc_l1, pc_l2, feat_l2, fps_idx1, fps_idx2)
